# baseline (speedup 1.0000x reference)
import jax
import jax.numpy as jnp
from jax.experimental import pallas as pl
from jax.experimental.pallas import tpu as pltpu

_B = 4096
_H = 1024
_BB = 512


def _dot_t(a, w):
    return jax.lax.dot_general(
        a, w, (((1,), (0,)), ((), ())), preferred_element_type=jnp.float32
    )


_NCHUNK = 2


def _lstm_body(x_ref, hx_ref, cx_ref, wxt_ref, wtf_ref, wcf_ref, wtu_ref,
               wcu_ref, wth_ref, wch_ref, bxt_ref, btf_ref, bcf_ref, btu_ref,
               bcu_ref, bth_ref, bch_ref, hy_ref, cy_ref):
    cb = _BB // _NCHUNK
    for c in range(_NCHUNK):
        rows = slice(c * cb, (c + 1) * cb)
        x = x_ref[rows, :]
        hx = hx_ref[rows, :]
        cx = cx_ref[rows, :]
        t = jnp.tanh(_dot_t(x, wxt_ref[...]) + bxt_ref[...]) + hx
        f = jax.nn.sigmoid(
            _dot_t(t, wtf_ref[...]) + _dot_t(cx, wcf_ref[...])
            + (btf_ref[...] + bcf_ref[...])
        )
        u = jax.nn.sigmoid(
            _dot_t(t, wtu_ref[...]) + _dot_t(cx, wcu_ref[...])
            + (btu_ref[...] + bcu_ref[...])
        ) * t
        cy = jnp.tanh(f * cx + u)
        hy = jnp.tanh(
            jax.nn.sigmoid(
                _dot_t(t, wth_ref[...]) + _dot_t(cy, wch_ref[...])
                + (bth_ref[...] + bch_ref[...])
            ) * cy
        )
        hy_ref[rows, :] = hy
        cy_ref[rows, :] = cy


def kernel(x, hx, cx, W_xt, W_tf, W_cf, W_tu, W_cu, W_th, W_ch,
           b_xt, b_tf, b_cf, b_tu, b_cu, b_th, b_ch):
    act_spec = pl.BlockSpec((_BB, _H), lambda i: (i, 0))
    w_spec = pl.BlockSpec((_H, _H), lambda i: (0, 0))
    b_spec = pl.BlockSpec((1, _H), lambda i: (0, 0))
    out = pl.pallas_call(
        _lstm_body,
        grid=(_B // _BB,),
        in_specs=[act_spec] * 3 + [w_spec] * 7 + [b_spec] * 7,
        out_specs=[
            pl.BlockSpec((_BB, _H), lambda i: (i, 0)),
            pl.BlockSpec((_BB, _H), lambda i: (i, 0)),
        ],
        out_shape=[
            jax.ShapeDtypeStruct((_B, _H), jnp.float32),
            jax.ShapeDtypeStruct((_B, _H), jnp.float32),
        ],
        compiler_params=pltpu.CompilerParams(
            dimension_semantics=("parallel",),
            vmem_limit_bytes=56 * 1024 * 1024,
        ),
        name="fused_lstm_cell",
    )(x, hx, cx,
      W_xt.T.astype(jnp.bfloat16), W_tf.T.astype(jnp.bfloat16),
      W_cf.T.astype(jnp.bfloat16), W_tu.T.astype(jnp.bfloat16),
      W_cu.T.astype(jnp.bfloat16), W_th.T.astype(jnp.bfloat16),
      W_ch.T.astype(jnp.bfloat16),
      b_xt.reshape(1, _H), b_tf.reshape(1, _H), b_cf.reshape(1, _H),
      b_tu.reshape(1, _H), b_cu.reshape(1, _H), b_th.reshape(1, _H),
      b_ch.reshape(1, _H))
    return (out[0], out[1])


# speedup vs baseline: 1.0750x; 1.0750x over previous
import jax
import jax.numpy as jnp
from jax.experimental import pallas as pl
from jax.experimental.pallas import tpu as pltpu

_B = 4096
_H = 1024
_BB = 256
_NB = _B // _BB


def _dot_t(a, w):
    return jax.lax.dot_general(
        a, w, (((1,), (1,)), ((), ())), preferred_element_type=jnp.float32
    )


def _lstm_body(x_hbm, hx_hbm, cx_hbm, w0_hbm, w1_hbm, w2_hbm, w3_hbm, w4_hbm,
               w5_hbm, w6_hbm, bxt, btf, bcf, btu, bcu, bth, bch,
               hy_hbm, cy_hbm,
               wvm, xbuf, hxbuf, cxbuf, hybuf, cybuf,
               wsem, xsem, hxsem, cxsem, hysem, cysem):
    w_hbm = (w0_hbm, w1_hbm, w2_hbm, w3_hbm, w4_hbm, w5_hbm, w6_hbm)
    for j in range(7):
        pltpu.make_async_copy(w_hbm[j], wvm.at[j], wsem.at[j]).start()
    for s in range(2):
        pltpu.make_async_copy(
            x_hbm.at[pl.ds(s * _BB, _BB), :], xbuf.at[s], xsem.at[s]).start()
        pltpu.make_async_copy(
            hx_hbm.at[pl.ds(s * _BB, _BB), :], hxbuf.at[s], hxsem.at[s]).start()
        pltpu.make_async_copy(
            cx_hbm.at[pl.ds(s * _BB, _BB), :], cxbuf.at[s], cxsem.at[s]).start()

    def _wait_w(j):
        pltpu.make_async_copy(w_hbm[j], wvm.at[j], wsem.at[j]).wait()

    def _wait_acts(slot):
        pltpu.make_async_copy(xbuf.at[slot], xbuf.at[slot], xsem.at[slot]).wait()
        pltpu.make_async_copy(hxbuf.at[slot], hxbuf.at[slot], hxsem.at[slot]).wait()
        pltpu.make_async_copy(cxbuf.at[slot], cxbuf.at[slot], cxsem.at[slot]).wait()

    def _wait_outs(slot, i):
        pltpu.make_async_copy(
            hybuf.at[slot], hy_hbm.at[pl.ds(i * _BB, _BB), :],
            hysem.at[slot]).wait()
        pltpu.make_async_copy(
            cybuf.at[slot], cy_hbm.at[pl.ds(i * _BB, _BB), :],
            cysem.at[slot]).wait()

    def _compute(x, hx, cx):
        t = jnp.tanh(_dot_t(x, wvm[0]) + bxt[...]) + hx
        f = jax.nn.sigmoid(
            _dot_t(t, wvm[1]) + _dot_t(cx, wvm[2]) + (btf[...] + bcf[...])
        )
        u = jax.nn.sigmoid(
            _dot_t(t, wvm[3]) + _dot_t(cx, wvm[4]) + (btu[...] + bcu[...])
        ) * t
        cy = jnp.tanh(f * cx + u)
        hy = jnp.tanh(
            jax.nn.sigmoid(
                _dot_t(t, wvm[5]) + _dot_t(cy, wvm[6]) + (bth[...] + bch[...])
            ) * cy
        )
        return hy, cy

    def _emit_out(slot, i, hy, cy):
        hybuf[slot] = hy
        cybuf[slot] = cy
        pltpu.make_async_copy(
            hybuf.at[slot], hy_hbm.at[pl.ds(i * _BB, _BB), :],
            hysem.at[slot]).start()
        pltpu.make_async_copy(
            cybuf.at[slot], cy_hbm.at[pl.ds(i * _BB, _BB), :],
            cysem.at[slot]).start()

    def _prefetch(slot, i):
        pltpu.make_async_copy(
            x_hbm.at[pl.ds(i * _BB, _BB), :], xbuf.at[slot],
            xsem.at[slot]).start()
        pltpu.make_async_copy(
            hx_hbm.at[pl.ds(i * _BB, _BB), :], hxbuf.at[slot],
            hxsem.at[slot]).start()
        pltpu.make_async_copy(
            cx_hbm.at[pl.ds(i * _BB, _BB), :], cxbuf.at[slot],
            cxsem.at[slot]).start()

    _wait_acts(0)
    x0, hx0, cx0 = xbuf[0], hxbuf[0], cxbuf[0]
    _wait_w(0)
    t0 = jnp.tanh(_dot_t(x0, wvm[0]) + bxt[...]) + hx0
    _wait_w(1)
    _wait_w(2)
    f0 = jax.nn.sigmoid(
        _dot_t(t0, wvm[1]) + _dot_t(cx0, wvm[2]) + (btf[...] + bcf[...])
    )
    _wait_w(3)
    _wait_w(4)
    u0 = jax.nn.sigmoid(
        _dot_t(t0, wvm[3]) + _dot_t(cx0, wvm[4]) + (btu[...] + bcu[...])
    ) * t0
    cy0 = jnp.tanh(f0 * cx0 + u0)
    _wait_w(5)
    _wait_w(6)
    hy0 = jnp.tanh(
        jax.nn.sigmoid(
            _dot_t(t0, wvm[5]) + _dot_t(cy0, wvm[6]) + (bth[...] + bch[...])
        ) * cy0
    )
    _emit_out(0, 0, hy0, cy0)
    _prefetch(0, 2)

    _wait_acts(1)
    hy1, cy1 = _compute(xbuf[1], hxbuf[1], cxbuf[1])
    _emit_out(1, 1, hy1, cy1)
    _prefetch(1, 3)

    def step(it, carry):
        j0 = 2 + 2 * it
        j1 = j0 + 1
        _wait_acts(0)
        _wait_acts(1)
        _wait_outs(0, j0 - 2)
        _wait_outs(1, j1 - 2)
        hy_a, cy_a = _compute(xbuf[0], hxbuf[0], cxbuf[0])
        hy_b, cy_b = _compute(xbuf[1], hxbuf[1], cxbuf[1])
        _emit_out(0, j0, hy_a, cy_a)
        _emit_out(1, j1, hy_b, cy_b)
        _prefetch(0, jnp.minimum(j0 + 2, _NB - 2))
        _prefetch(1, jnp.minimum(j1 + 2, _NB - 1))
        return carry

    jax.lax.fori_loop(0, (_NB - 2) // 2, step, 0)

    _wait_outs(0, _NB - 2)
    _wait_outs(1, _NB - 1)
    _wait_acts(0)
    _wait_acts(1)


def kernel(x, hx, cx, W_xt, W_tf, W_cf, W_tu, W_cu, W_th, W_ch,
           b_xt, b_tf, b_cf, b_tu, b_cu, b_th, b_ch):
    any_spec = pl.BlockSpec(memory_space=pl.MemorySpace.ANY)
    vmem_spec = pl.BlockSpec(memory_space=pltpu.VMEM)
    out = pl.pallas_call(
        _lstm_body,
        in_specs=[any_spec] * 10 + [vmem_spec] * 7,
        out_specs=[any_spec, any_spec],
        out_shape=[
            jax.ShapeDtypeStruct((_B, _H), jnp.float32),
            jax.ShapeDtypeStruct((_B, _H), jnp.float32),
        ],
        scratch_shapes=[
            pltpu.VMEM((7, _H, _H), jnp.float32),
            pltpu.VMEM((2, _BB, _H), jnp.float32),
            pltpu.VMEM((2, _BB, _H), jnp.float32),
            pltpu.VMEM((2, _BB, _H), jnp.float32),
            pltpu.VMEM((2, _BB, _H), jnp.float32),
            pltpu.VMEM((2, _BB, _H), jnp.float32),
            pltpu.SemaphoreType.DMA((7,)),
            pltpu.SemaphoreType.DMA((2,)),
            pltpu.SemaphoreType.DMA((2,)),
            pltpu.SemaphoreType.DMA((2,)),
            pltpu.SemaphoreType.DMA((2,)),
            pltpu.SemaphoreType.DMA((2,)),
        ],
        compiler_params=pltpu.CompilerParams(
            vmem_limit_bytes=56 * 1024 * 1024,
        ),
        name="fused_lstm_cell_manual",
    )(x, hx, cx, W_xt, W_tf, W_cf, W_tu, W_cu, W_th, W_ch,
      b_xt.reshape(1, _H), b_tf.reshape(1, _H), b_cf.reshape(1, _H),
      b_tu.reshape(1, _H), b_cu.reshape(1, _H), b_th.reshape(1, _H),
      b_ch.reshape(1, _H))
    return (out[0], out[1])


# speedup vs baseline: 1.2323x; 1.1463x over previous
import jax
import jax.numpy as jnp
from jax.experimental import pallas as pl
from jax.experimental.pallas import tpu as pltpu

_B = 4096
_H = 1024
_BB = 256
_NB = _B // _BB
_NPAIR = (_NB - 2) // 2


def _dot_t(a, w):
    return jax.lax.dot_general(
        a, w, (((1,), (1,)), ((), ())), preferred_element_type=jnp.float32
    )


def _lstm_body(x_hbm, hx_hbm, cx_hbm, w0_hbm, w1_hbm, w2_hbm, w3_hbm, w4_hbm,
               w5_hbm, w6_hbm, bxt, btf, bcf, btu, bcu, bth, bch,
               hy_hbm, cy_hbm,
               wvm, xbuf, hxbuf, cxbuf, hybuf, cybuf,
               wsem, xsem, hxsem, cxsem, hysem, cysem):
    w_hbm = (w0_hbm, w1_hbm, w2_hbm, w3_hbm, w4_hbm, w5_hbm, w6_hbm)
    for j in range(7):
        pltpu.make_async_copy(w_hbm[j], wvm.at[j], wsem.at[j]).start()

    def _prefetch(slot, i):
        pltpu.make_async_copy(
            x_hbm.at[pl.ds(i * _BB, _BB), :], xbuf.at[slot],
            xsem.at[slot]).start()
        pltpu.make_async_copy(
            hx_hbm.at[pl.ds(i * _BB, _BB), :], hxbuf.at[slot],
            hxsem.at[slot]).start()
        pltpu.make_async_copy(
            cx_hbm.at[pl.ds(i * _BB, _BB), :], cxbuf.at[slot],
            cxsem.at[slot]).start()

    _prefetch(0, 0)
    _prefetch(1, 1)

    def _wait_w(j):
        pltpu.make_async_copy(w_hbm[j], wvm.at[j], wsem.at[j]).wait()

    def _wait_acts(slot):
        pltpu.make_async_copy(xbuf.at[slot], xbuf.at[slot], xsem.at[slot]).wait()
        pltpu.make_async_copy(hxbuf.at[slot], hxbuf.at[slot], hxsem.at[slot]).wait()
        pltpu.make_async_copy(cxbuf.at[slot], cxbuf.at[slot], cxsem.at[slot]).wait()

    def _wait_outs(slot, i):
        pltpu.make_async_copy(
            hybuf.at[slot], hy_hbm.at[pl.ds(i * _BB, _BB), :],
            hysem.at[slot]).wait()
        pltpu.make_async_copy(
            cybuf.at[slot], cy_hbm.at[pl.ds(i * _BB, _BB), :],
            cysem.at[slot]).wait()

    def _compute(x, hx, cx):
        t = jnp.tanh(_dot_t(x, wvm[0]) + bxt[...]) + hx
        f = jax.nn.sigmoid(
            _dot_t(t, wvm[1]) + _dot_t(cx, wvm[2]) + (btf[...] + bcf[...])
        )
        u = jax.nn.sigmoid(
            _dot_t(t, wvm[3]) + _dot_t(cx, wvm[4]) + (btu[...] + bcu[...])
        ) * t
        cy = jnp.tanh(f * cx + u)
        hy = jnp.tanh(
            jax.nn.sigmoid(
                _dot_t(t, wvm[5]) + _dot_t(cy, wvm[6]) + (bth[...] + bch[...])
            ) * cy
        )
        return hy, cy

    def _emit_out(slot, i, hy, cy):
        hybuf[slot] = hy
        cybuf[slot] = cy
        pltpu.make_async_copy(
            hybuf.at[slot], hy_hbm.at[pl.ds(i * _BB, _BB), :],
            hysem.at[slot]).start()
        pltpu.make_async_copy(
            cybuf.at[slot], cy_hbm.at[pl.ds(i * _BB, _BB), :],
            cysem.at[slot]).start()

    _wait_acts(0)
    x0, hx0, cx0 = xbuf[0], hxbuf[0], cxbuf[0]
    _wait_w(0)
    t0 = jnp.tanh(_dot_t(x0, wvm[0]) + bxt[...]) + hx0
    _wait_w(1)
    _wait_w(2)
    f0 = jax.nn.sigmoid(
        _dot_t(t0, wvm[1]) + _dot_t(cx0, wvm[2]) + (btf[...] + bcf[...])
    )
    _wait_w(3)
    _wait_w(4)
    u0 = jax.nn.sigmoid(
        _dot_t(t0, wvm[3]) + _dot_t(cx0, wvm[4]) + (btu[...] + bcu[...])
    ) * t0
    cy0 = jnp.tanh(f0 * cx0 + u0)
    _wait_w(5)
    _wait_w(6)
    hy0 = jnp.tanh(
        jax.nn.sigmoid(
            _dot_t(t0, wvm[5]) + _dot_t(cy0, wvm[6]) + (bth[...] + bch[...])
        ) * cy0
    )
    _emit_out(0, 0, hy0, cy0)
    _prefetch(2, 2)

    _wait_acts(1)
    hy1, cy1 = _compute(xbuf[1], hxbuf[1], cxbuf[1])
    _emit_out(1, 1, hy1, cy1)
    _prefetch(3, 3)

    def step(it, carry):
        j0 = 2 + 2 * it
        cur = 2 * jax.lax.rem(it + 1, 2)
        nxt = 2 - cur
        _prefetch(nxt, jnp.minimum(j0 + 2, _NB - 2))
        _prefetch(nxt + 1, jnp.minimum(j0 + 3, _NB - 1))
        _wait_acts(cur)
        _wait_acts(cur + 1)
        hy_a, cy_a = _compute(xbuf[cur], hxbuf[cur], cxbuf[cur])
        hy_b, cy_b = _compute(xbuf[cur + 1], hxbuf[cur + 1], cxbuf[cur + 1])
        _wait_outs(0, j0)
        _wait_outs(1, j0 + 1)
        _emit_out(0, j0, hy_a, cy_a)
        _emit_out(1, j0 + 1, hy_b, cy_b)
        return carry

    jax.lax.fori_loop(0, _NPAIR, step, 0)

    _wait_outs(0, _NB - 2)
    _wait_outs(1, _NB - 1)
    nxt_last = 2 - 2 * (_NPAIR % 2)
    _wait_acts(nxt_last)
    _wait_acts(nxt_last + 1)


def kernel(x, hx, cx, W_xt, W_tf, W_cf, W_tu, W_cu, W_th, W_ch,
           b_xt, b_tf, b_cf, b_tu, b_cu, b_th, b_ch):
    any_spec = pl.BlockSpec(memory_space=pl.MemorySpace.ANY)
    vmem_spec = pl.BlockSpec(memory_space=pltpu.VMEM)
    out = pl.pallas_call(
        _lstm_body,
        in_specs=[any_spec] * 10 + [vmem_spec] * 7,
        out_specs=[any_spec, any_spec],
        out_shape=[
            jax.ShapeDtypeStruct((_B, _H), jnp.float32),
            jax.ShapeDtypeStruct((_B, _H), jnp.float32),
        ],
        scratch_shapes=[
            pltpu.VMEM((7, _H, _H), jnp.float32),
            pltpu.VMEM((4, _BB, _H), jnp.float32),
            pltpu.VMEM((4, _BB, _H), jnp.float32),
            pltpu.VMEM((4, _BB, _H), jnp.float32),
            pltpu.VMEM((2, _BB, _H), jnp.float32),
            pltpu.VMEM((2, _BB, _H), jnp.float32),
            pltpu.SemaphoreType.DMA((7,)),
            pltpu.SemaphoreType.DMA((4,)),
            pltpu.SemaphoreType.DMA((4,)),
            pltpu.SemaphoreType.DMA((4,)),
            pltpu.SemaphoreType.DMA((2,)),
            pltpu.SemaphoreType.DMA((2,)),
        ],
        compiler_params=pltpu.CompilerParams(
            vmem_limit_bytes=65024 * 1024,
        ),
        name="fused_lstm_cell_manual",
    )(x, hx, cx, W_xt, W_tf, W_cf, W_tu, W_cu, W_th, W_ch,
      b_xt.reshape(1, _H), b_tf.reshape(1, _H), b_cf.reshape(1, _H),
      b_tu.reshape(1, _H), b_cu.reshape(1, _H), b_th.reshape(1, _H),
      b_ch.reshape(1, _H))
    return (out[0], out[1])
